# Initial kernel scaffold; baseline (speedup 1.0000x reference)
import jax
import jax.numpy as jnp
from jax.experimental import pallas as pl


def kernel(term_embs, pred_embs, inv_pred_embs, signs, head_idx, tail_idx, W1, b1, W2, b2):
    raise NotImplementedError("write your pallas kernel here")


# speedup vs baseline: 2.5817x; 2.5817x over previous
import functools

import jax
import jax.numpy as jnp
from jax.experimental import pallas as pl
from jax.experimental.pallas import tpu as pltpu

_EPS = 0.1


def _agg_body(a_ref, oht_ref, ohh_ref, term_ref, pred_ref, ipred_ref, out_ref):
    acc = jnp.dot(a_ref[...], term_ref[...], preferred_element_type=jnp.float32)
    acc += jnp.dot(oht_ref[...], pred_ref[...], preferred_element_type=jnp.float32)
    acc += jnp.dot(ohh_ref[...], ipred_ref[...], preferred_element_type=jnp.float32)
    out_ref[...] = acc


def _mlp_body(x_ref, w1_ref, b1_ref, w2_ref, b2_ref, out_ref):
    x = x_ref[...]
    h = jnp.dot(x, w1_ref[...], preferred_element_type=jnp.float32) + b1_ref[...]
    h = jnp.maximum(h, 0.0)
    out_ref[...] = (
        jnp.dot(h, w2_ref[...], preferred_element_type=jnp.float32) + b2_ref[...]
    )


@functools.partial(jax.jit, static_argnames=())
def kernel(term_embs, pred_embs, inv_pred_embs, signs, head_idx, tail_idx,
           W1, b1, W2, b2):
    T, B, D = term_embs.shape
    E = pred_embs.shape[0]
    H = W1.shape[1]
    N = B * D

    f32 = jnp.float32
    t_iota = jnp.arange(T, dtype=jnp.int32)[:, None]
    oht = (tail_idx[None, :].astype(jnp.int32) == t_iota).astype(f32)
    ohh = (head_idx[None, :].astype(jnp.int32) == t_iota).astype(f32)
    oht_s = oht * signs[None, :]
    ohh_s = ohh * signs[None, :]
    a_mat = (oht_s @ ohh.T + ohh_s @ oht.T
             + _EPS * jnp.eye(T, dtype=f32))

    term2 = term_embs.reshape(T, N)
    pred2 = pred_embs.reshape(E, N)
    ipred2 = inv_pred_embs.reshape(E, N)

    CT = 32768
    nc = N // CT
    agg = pl.pallas_call(
        _agg_body,
        grid=(nc,),
        in_specs=[
            pl.BlockSpec((T, T), lambda i: (0, 0)),
            pl.BlockSpec((T, E), lambda i: (0, 0)),
            pl.BlockSpec((T, E), lambda i: (0, 0)),
            pl.BlockSpec((T, CT), lambda i: (0, i)),
            pl.BlockSpec((E, CT), lambda i: (0, i)),
            pl.BlockSpec((E, CT), lambda i: (0, i)),
        ],
        out_specs=pl.BlockSpec((T, CT), lambda i: (0, i)),
        out_shape=jax.ShapeDtypeStruct((T, N), f32),
        compiler_params=pltpu.CompilerParams(
            dimension_semantics=("parallel",)),
    )(a_mat, oht_s, ohh_s, term2, pred2, ipred2)

    rows = T * B
    RT = 2048
    nr = rows // RT
    agg_rows = agg.reshape(rows, D)
    out = pl.pallas_call(
        _mlp_body,
        grid=(nr,),
        in_specs=[
            pl.BlockSpec((RT, D), lambda i: (i, 0)),
            pl.BlockSpec((D, H), lambda i: (0, 0)),
            pl.BlockSpec((1, H), lambda i: (0, 0)),
            pl.BlockSpec((H, D), lambda i: (0, 0)),
            pl.BlockSpec((1, D), lambda i: (0, 0)),
        ],
        out_specs=pl.BlockSpec((RT, D), lambda i: (i, 0)),
        out_shape=jax.ShapeDtypeStruct((rows, D), f32),
        compiler_params=pltpu.CompilerParams(
            dimension_semantics=("parallel",)),
    )(agg_rows, W1, b1.reshape(1, H), W2, b2.reshape(1, D))

    return out.reshape(T, B, D)


# speedup vs baseline: 10.6508x; 4.1255x over previous
import functools

import jax
import jax.numpy as jnp
from jax.experimental import pallas as pl
from jax.experimental.pallas import tpu as pltpu

_EPS = 0.1


def _fused_body(head_ref, tail_ref, signs_ref, term_ref, pred_ref, ipred_ref,
                w1_ref, b1_ref, w2_ref, b2_ref, out_ref, acc_ref):
    E = pred_ref.shape[0]
    T, BT, D = term_ref.shape
    H = w1_ref.shape[1]

    acc_ref[...] = _EPS * term_ref[...]
    for e in range(E):
        h = head_ref[e]
        t = tail_ref[e]
        s = signs_ref[e]
        acc_ref[t] += s * (term_ref[h] + pred_ref[e])
        acc_ref[h] += s * (term_ref[t] + ipred_ref[e])

    x = acc_ref[...].reshape(T * BT, D)
    hidden = jnp.dot(x, w1_ref[...], preferred_element_type=jnp.float32)
    hidden = jnp.maximum(hidden + b1_ref[...], 0.0)
    y = jnp.dot(hidden, w2_ref[...], preferred_element_type=jnp.float32)
    y = y + b2_ref[...]
    out_ref[...] = y.reshape(T, BT, D)


@functools.partial(jax.jit, static_argnames=())
def kernel(term_embs, pred_embs, inv_pred_embs, signs, head_idx, tail_idx,
           W1, b1, W2, b2):
    T, B, D = term_embs.shape
    E = pred_embs.shape[0]
    H = W1.shape[1]

    BT = 256
    nb = B // BT

    smem = pl.BlockSpec(memory_space=pltpu.SMEM)
    out = pl.pallas_call(
        _fused_body,
        grid=(nb,),
        in_specs=[
            smem,
            smem,
            smem,
            pl.BlockSpec((T, BT, D), lambda i: (0, i, 0)),
            pl.BlockSpec((E, BT, D), lambda i: (0, i, 0)),
            pl.BlockSpec((E, BT, D), lambda i: (0, i, 0)),
            pl.BlockSpec((D, H), lambda i: (0, 0)),
            pl.BlockSpec((1, H), lambda i: (0, 0)),
            pl.BlockSpec((H, D), lambda i: (0, 0)),
            pl.BlockSpec((1, D), lambda i: (0, 0)),
        ],
        out_specs=pl.BlockSpec((T, BT, D), lambda i: (0, i, 0)),
        out_shape=jax.ShapeDtypeStruct((T, B, D), jnp.float32),
        scratch_shapes=[pltpu.VMEM((T, BT, D), jnp.float32)],
        compiler_params=pltpu.CompilerParams(
            dimension_semantics=("parallel",)),
    )(head_idx.astype(jnp.int32), tail_idx.astype(jnp.int32), signs,
      term_embs, pred_embs, inv_pred_embs,
      W1, b1.reshape(1, H), W2, b2.reshape(1, D))

    return out
